# baseline (speedup 1.0000x reference)
import functools

import jax
import jax.numpy as jnp
import numpy as np
from jax import lax
from jax.experimental import pallas as pl
from jax.experimental.pallas import tpu as pltpu
from jax.experimental.pallas import tpu_sc as plsc

B, L = 16, 8
H = 256
E = 128
MN = 32
MS = 65536
NR = 2
NQ = L * NR
G = 4
CHUNK = 16384
CS = CHUNK // G
HCS = CS // 2
NCHUNK = MS // CHUNK
INV_SQRT_MN = 1.0 / float(np.sqrt(32.0))
GQ = G * NQ
GM = G * MN


def _sc_gather(aw, dw, past2d, ac_flat, dn_flat, xw_flat):
    mesh = plsc.VectorSubcoreMesh(core_axis_name="c", subcore_axis_name="s")

    @functools.partial(
        pl.kernel,
        mesh=mesh,
        out_type=[
            jax.ShapeDtypeStruct((B * L, E), jnp.float32),
            jax.ShapeDtypeStruct((B * L, E), jnp.float32),
            jax.ShapeDtypeStruct((B * L, 128), jnp.float32),
        ],
        scratch_types=[
            pltpu.VMEM((16,), jnp.int32),
            pltpu.VMEM((16,), jnp.int32),
            pltpu.VMEM((16, E), jnp.float32),
            pltpu.SemaphoreType.DMA,
        ],
    )
    def k(aw_h, dw_h, past_h, ac_h, dn_h, xw_h, ae_o, de_o, old_o,
          idx_v, idx2_v, rows_v, sem):
        c = lax.axis_index("c")
        s = lax.axis_index("s")
        wid = s * 2 + c
        grp = wid // 8
        base = pl.multiple_of((wid % 8) * 16, 16)

        @pl.when(grp == 0)
        def _():
            pltpu.sync_copy(ac_h.at[pl.ds(base, 16)], idx_v)
            pltpu.async_copy(aw_h.at[idx_v], rows_v, sem).wait()
            pltpu.sync_copy(rows_v, ae_o.at[pl.ds(base, 16)])

        @pl.when(grp == 1)
        def _():
            pltpu.sync_copy(dn_h.at[pl.ds(base, 16)], idx_v)
            pltpu.async_copy(dw_h.at[idx_v], rows_v, sem).wait()
            pltpu.sync_copy(rows_v, de_o.at[pl.ds(base, 16)])

        @pl.when(grp == 2)
        def _():
            pltpu.sync_copy(xw_h.at[pl.ds(base, 16)], idx_v)
            xv = idx_v[...]
            half = lax.shift_right_logical(
                lax.broadcasted_iota(jnp.int32, (16,), 0), 3)
            brow = (base // 8) + half
            idx2_v[...] = lax.shift_right_logical(xv + brow * MS, 2)
            pltpu.async_copy(past_h.at[idx2_v], rows_v, sem).wait()
            pltpu.sync_copy(rows_v, old_o.at[pl.ds(base, 16)])

    return k(aw, dw, past2d, ac_flat, dn_flat, xw_flat)


def _stream_body(past_ref, state_ref, wt_ref, bt_ref, wrk0_ref, wrk1_ref,
                 accg_ref, seg_ref, hread_ref, keys_ref,
                 bd_s, acc_a, acc_b, se_s):
    i = pl.program_id(1)

    @pl.when(i == 0)
    def _():
        a_state = state_ref[0, 0]
        t_state = state_ref[0, 1]
        hread = t_state + jax.nn.gelu(
            jnp.dot(a_state, wt_ref[...]) + bt_ref[...])
        hread_ref[0] = hread
        keys = jnp.concatenate(
            [jnp.dot(hread, wrk0_ref[...]),
             jnp.dot(hread, wrk1_ref[...])], axis=0)
        keys_ref[0] = keys
        kt = jnp.transpose(keys)
        t1 = jnp.concatenate([kt] * G, axis=0)
        t2 = jnp.concatenate([t1] * G, axis=1)
        rowg = lax.broadcasted_iota(jnp.int32, (GM, GQ), 0) // MN
        colg = lax.broadcasted_iota(jnp.int32, (GM, GQ), 1) // NQ
        bd_s[...] = jnp.where(rowg == colg, t2, 0.0)
        acc_a[...] = jnp.zeros_like(acc_a)
        acc_b[...] = jnp.zeros_like(acc_b)
        se_s[...] = jnp.zeros_like(se_s)

    bd = bd_s[...]
    ca = past_ref[0:HCS]
    cb = past_ref[HCS:CS]
    la = jnp.dot(ca, bd)
    lb = jnp.dot(cb, bd)
    pa = jnp.exp(la * INV_SQRT_MN)
    pb = jnp.exp(lb * INV_SQRT_MN)
    se_s[...] += (jnp.sum(pa, axis=0, keepdims=True)
                  + jnp.sum(pb, axis=0, keepdims=True))
    acc_a[...] += lax.dot_general(pa, ca, (((0,), (0,)), ((), ())))
    acc_b[...] += lax.dot_general(pb, cb, (((0,), (0,)), ((), ())))

    @pl.when(i == NCHUNK - 1)
    def _():
        accg_ref[0] = acc_a[...] + acc_b[...]
        seg_ref[0] = se_s[...]


def _stream_call(past2d, state, Wt, bt2, Wrk0, Wrk1, interpret=False):
    const = lambda *blk: pl.BlockSpec(blk, lambda b, i: (0,) * len(blk))
    perb = lambda *blk: pl.BlockSpec(blk, lambda b, i: (b,) + (0,) * (len(blk) - 1))
    return pl.pallas_call(
        _stream_body,
        grid=(B, NCHUNK),
        in_specs=[
            pl.BlockSpec((CS, GM), lambda b, i: (b * NCHUNK + i, 0)),
            perb(1, 2, L, H),
            const(H, H),
            const(1, H),
            const(H, MN),
            const(H, MN),
        ],
        out_specs=[
            perb(1, GQ, GM),
            perb(1, 1, GQ),
            perb(1, L, H),
            perb(1, NQ, MN),
        ],
        out_shape=[
            jax.ShapeDtypeStruct((B, GQ, GM), jnp.float32),
            jax.ShapeDtypeStruct((B, 1, GQ), jnp.float32),
            jax.ShapeDtypeStruct((B, L, H), jnp.float32),
            jax.ShapeDtypeStruct((B, NQ, MN), jnp.float32),
        ],
        scratch_shapes=[
            pltpu.VMEM((GM, GQ), jnp.float32),
            pltpu.VMEM((GQ, GM), jnp.float32),
            pltpu.VMEM((GQ, GM), jnp.float32),
            pltpu.VMEM((1, GQ), jnp.float32),
        ],
        compiler_params=pltpu.CompilerParams(
            dimension_semantics=("parallel", "arbitrary")),
        interpret=interpret,
    )(past2d, state, Wt, bt2, Wrk0, Wrk1)


def _epi_body(accg_ref, seg_ref, hread_ref, keys_ref, pstate_ref,
              ae_ref, de_ref, old_ref, rw_ref, noise_ref, xwr_ref, xwc_ref,
              wt_ref, bt_ref, wwrh_ref, wwre_ref, wo1_ref, wo2a_ref, wo2b_ref,
              y_ref):
    a_ps = pstate_ref[0, 0]
    t_ps = pstate_ref[0, 1]
    hwrite = t_ps + jax.nn.gelu(jnp.dot(a_ps, wt_ref[...]) + bt_ref[...])
    ard = ae_ref[0] + (rw_ref[0] + noise_ref[0]) + de_ref[0]
    v = jnp.dot(hwrite, wwrh_ref[...]) + jnp.dot(ard, wwre_ref[...])
    keys = keys_ref[0]
    wide = old_ref[0]
    sub = jnp.bitwise_and(xwc_ref[0], G - 1)
    old = jnp.zeros((L, MN), jnp.float32)
    for g in range(G):
        old = old + jnp.where(sub == g, wide[:, g * MN:(g + 1) * MN], 0.0)
    lo = lax.dot_general(old, keys, (((1,), (1,)), ((), ()))) * INV_SQRT_MN
    ln = lax.dot_general(v, keys, (((1,), (1,)), ((), ()))) * INV_SQRT_MN
    eq = xwc_ref[0] == xwr_ref[0]
    later = (lax.broadcasted_iota(jnp.int32, (L, L), 1)
             > lax.broadcasted_iota(jnp.int32, (L, L), 0))
    dup = jnp.any(eq & later, axis=1, keepdims=True)
    valid = jnp.where(dup, 0.0, 1.0)
    elo = jnp.exp(lo) * valid
    eln = jnp.exp(ln) * valid
    seg = seg_ref[0]
    accg = accg_ref[0]
    se16 = jnp.zeros((1, NQ), jnp.float32)
    acc16 = jnp.zeros((NQ, MN), jnp.float32)
    for g in range(G):
        se16 = se16 + seg[:, g * NQ:(g + 1) * NQ]
        acc16 = acc16 + accg[g * NQ:(g + 1) * NQ, g * MN:(g + 1) * MN]
    se = se16 + jnp.sum(eln - elo, axis=0, keepdims=True)
    acc = (acc16
           + lax.dot_general(eln, v, (((0,), (0,)), ((), ())))
           - lax.dot_general(elo, old, (((0,), (0,)), ((), ()))))
    eye = (lax.broadcasted_iota(jnp.int32, (NQ, NQ), 0)
           == lax.broadcasted_iota(jnp.int32, (NQ, NQ), 1))
    se_col = jnp.sum(jnp.where(eye, se, 0.0), axis=1, keepdims=True)
    reads = acc / se_col
    y = (jnp.dot(hread_ref[0], wo1_ref[...])
         + jnp.dot(reads[0:L], wo2a_ref[...])
         + jnp.dot(reads[L:NQ], wo2b_ref[...]))
    y_ref[0] = y


def _epi_call(accg, seg, hread, keys, pstate, ae3, de3, old3, rw_col, noise3,
              xw_row, xw_col, Wt, bt2, Wwrh, Wwre, Wo1, Wo2a, Wo2b,
              interpret=False):
    const = lambda *blk: pl.BlockSpec(blk, lambda b: (0,) * len(blk))
    perb = lambda *blk: pl.BlockSpec(blk, lambda b: (b,) + (0,) * (len(blk) - 1))
    return pl.pallas_call(
        _epi_body,
        grid=(B,),
        in_specs=[
            perb(1, GQ, GM),
            perb(1, 1, GQ),
            perb(1, L, H),
            perb(1, NQ, MN),
            perb(1, 2, L, H),
            perb(1, L, E),
            perb(1, L, E),
            perb(1, L, 128),
            perb(1, L, 1),
            perb(1, L, E),
            perb(1, 1, L),
            perb(1, L, 1),
            const(H, H),
            const(1, H),
            const(H, MN),
            const(E, MN),
            const(H, H),
            const(MN, H),
            const(MN, H),
        ],
        out_specs=perb(1, L, H),
        out_shape=jax.ShapeDtypeStruct((B, L, H), jnp.float32),
        compiler_params=pltpu.CompilerParams(
            dimension_semantics=("arbitrary",)),
        interpret=interpret,
    )(accg, seg, hread, keys, pstate, ae3, de3, old3, rw_col, noise3,
      xw_row, xw_col, Wt, bt2, Wwrh, Wwre, Wo1, Wo2a, Wo2b)


def kernel(state, pstate, ac, rw, dn, x_w, step, params, past,
           Wt, bt, aw, dw, W_wr, W_rk, W_o):
    noise = 0.001 * jax.random.normal(
        jax.random.key(1), (B, L, E), dtype=jnp.float32)
    past2d = past.reshape(B * MS // G, GM)
    bt2 = bt.reshape(1, H)
    ae_f, de_f, old_f = _sc_gather(
        aw, dw, past2d,
        ac.reshape(B * L), dn.reshape(B * L), x_w.reshape(B * L))
    accg, seg, hread, keys = _stream_call(
        past2d, state, Wt, bt2, W_rk[:H, 0:MN], W_rk[:H, MN:2 * MN])
    y = _epi_call(
        accg, seg, hread, keys, pstate,
        ae_f.reshape(B, L, E), de_f.reshape(B, L, E),
        old_f.reshape(B, L, 128),
        rw.reshape(B, L, 1), noise,
        x_w.reshape(B, 1, L), x_w.reshape(B, L, 1),
        Wt, bt2, W_wr[:H], W_wr[H:],
        W_o[:H], W_o[H + E:H + E + MN], W_o[H + E + MN:])
    return y


# speedup vs baseline: 1.0381x; 1.0381x over previous
import functools

import jax
import jax.numpy as jnp
import numpy as np
from jax import lax
from jax.experimental import pallas as pl
from jax.experimental.pallas import tpu as pltpu
from jax.experimental.pallas import tpu_sc as plsc

B, L = 16, 8
H = 256
E = 128
MN = 32
MS = 65536
NR = 2
NQ = L * NR
G = 4
CHUNK = 32768
CS = CHUNK // G
HCS = CS // 2
NCHUNK = MS // CHUNK
INV_SQRT_MN = 1.0 / float(np.sqrt(32.0))
GQ = G * NQ
GM = G * MN


def _sc_gather(aw, dw, past2d, ac_flat, dn_flat, xw_flat):
    mesh = plsc.VectorSubcoreMesh(core_axis_name="c", subcore_axis_name="s")

    @functools.partial(
        pl.kernel,
        mesh=mesh,
        out_type=[
            jax.ShapeDtypeStruct((B * L, E), jnp.float32),
            jax.ShapeDtypeStruct((B * L, E), jnp.float32),
            jax.ShapeDtypeStruct((B * L, 128), jnp.float32),
        ],
        scratch_types=[
            pltpu.VMEM((16,), jnp.int32),
            pltpu.VMEM((16,), jnp.int32),
            pltpu.VMEM((16, E), jnp.float32),
            pltpu.SemaphoreType.DMA,
        ],
    )
    def k(aw_h, dw_h, past_h, ac_h, dn_h, xw_h, ae_o, de_o, old_o,
          idx_v, idx2_v, rows_v, sem):
        c = lax.axis_index("c")
        s = lax.axis_index("s")
        wid = s * 2 + c
        grp = wid // 8
        base = pl.multiple_of((wid % 8) * 16, 16)

        @pl.when(grp == 0)
        def _():
            pltpu.sync_copy(ac_h.at[pl.ds(base, 16)], idx_v)
            pltpu.async_copy(aw_h.at[idx_v], rows_v, sem).wait()
            pltpu.sync_copy(rows_v, ae_o.at[pl.ds(base, 16)])

        @pl.when(grp == 1)
        def _():
            pltpu.sync_copy(dn_h.at[pl.ds(base, 16)], idx_v)
            pltpu.async_copy(dw_h.at[idx_v], rows_v, sem).wait()
            pltpu.sync_copy(rows_v, de_o.at[pl.ds(base, 16)])

        @pl.when(grp == 2)
        def _():
            pltpu.sync_copy(xw_h.at[pl.ds(base, 16)], idx_v)
            xv = idx_v[...]
            half = lax.shift_right_logical(
                lax.broadcasted_iota(jnp.int32, (16,), 0), 3)
            brow = (base // 8) + half
            idx2_v[...] = lax.shift_right_logical(xv + brow * MS, 2)
            pltpu.async_copy(past_h.at[idx2_v], rows_v, sem).wait()
            pltpu.sync_copy(rows_v, old_o.at[pl.ds(base, 16)])

    return k(aw, dw, past2d, ac_flat, dn_flat, xw_flat)


def _stream_body(past_ref, state_ref, wt_ref, bt_ref, wrk0_ref, wrk1_ref,
                 accg_ref, seg_ref, hread_ref, keys_ref,
                 bd_s, acc_a, acc_b, se_s):
    i = pl.program_id(1)

    @pl.when(i == 0)
    def _():
        a_state = state_ref[0, 0]
        t_state = state_ref[0, 1]
        hread = t_state + jax.nn.gelu(
            jnp.dot(a_state, wt_ref[...]) + bt_ref[...])
        hread_ref[0] = hread
        keys = jnp.concatenate(
            [jnp.dot(hread, wrk0_ref[...]),
             jnp.dot(hread, wrk1_ref[...])], axis=0)
        keys_ref[0] = keys
        kt = jnp.transpose(keys)
        t1 = jnp.concatenate([kt] * G, axis=0)
        t2 = jnp.concatenate([t1] * G, axis=1)
        rowg = lax.broadcasted_iota(jnp.int32, (GM, GQ), 0) // MN
        colg = lax.broadcasted_iota(jnp.int32, (GM, GQ), 1) // NQ
        bd_s[...] = jnp.where(rowg == colg, t2, 0.0)
        acc_a[...] = jnp.zeros_like(acc_a)
        acc_b[...] = jnp.zeros_like(acc_b)
        se_s[...] = jnp.zeros_like(se_s)

    bd = bd_s[...]
    ca = past_ref[0:HCS]
    cb = past_ref[HCS:CS]
    la = jnp.dot(ca, bd)
    lb = jnp.dot(cb, bd)
    pa = jnp.exp(la * INV_SQRT_MN)
    pb = jnp.exp(lb * INV_SQRT_MN)
    se_s[...] += (jnp.sum(pa, axis=0, keepdims=True)
                  + jnp.sum(pb, axis=0, keepdims=True))
    acc_a[...] += lax.dot_general(pa, ca, (((0,), (0,)), ((), ())))
    acc_b[...] += lax.dot_general(pb, cb, (((0,), (0,)), ((), ())))

    @pl.when(i == NCHUNK - 1)
    def _():
        accg_ref[0] = acc_a[...] + acc_b[...]
        seg_ref[0] = se_s[...]


def _stream_call(past2d, state, Wt, bt2, Wrk0, Wrk1, interpret=False):
    const = lambda *blk: pl.BlockSpec(blk, lambda b, i: (0,) * len(blk))
    perb = lambda *blk: pl.BlockSpec(blk, lambda b, i: (b,) + (0,) * (len(blk) - 1))
    return pl.pallas_call(
        _stream_body,
        grid=(B, NCHUNK),
        in_specs=[
            pl.BlockSpec((CS, GM), lambda b, i: (b * NCHUNK + i, 0)),
            perb(1, 2, L, H),
            const(H, H),
            const(1, H),
            const(H, MN),
            const(H, MN),
        ],
        out_specs=[
            perb(1, GQ, GM),
            perb(1, 1, GQ),
            perb(1, L, H),
            perb(1, NQ, MN),
        ],
        out_shape=[
            jax.ShapeDtypeStruct((B, GQ, GM), jnp.float32),
            jax.ShapeDtypeStruct((B, 1, GQ), jnp.float32),
            jax.ShapeDtypeStruct((B, L, H), jnp.float32),
            jax.ShapeDtypeStruct((B, NQ, MN), jnp.float32),
        ],
        scratch_shapes=[
            pltpu.VMEM((GM, GQ), jnp.float32),
            pltpu.VMEM((GQ, GM), jnp.float32),
            pltpu.VMEM((GQ, GM), jnp.float32),
            pltpu.VMEM((1, GQ), jnp.float32),
        ],
        compiler_params=pltpu.CompilerParams(
            dimension_semantics=("parallel", "arbitrary")),
        interpret=interpret,
    )(past2d, state, Wt, bt2, Wrk0, Wrk1)


def _epi_body(accg_ref, seg_ref, hread_ref, keys_ref, pstate_ref,
              ae_ref, de_ref, old_ref, rw_ref, noise_ref, xwr_ref, xwc_ref,
              wt_ref, bt_ref, wwrh_ref, wwre_ref, wo1_ref, wo2a_ref, wo2b_ref,
              y_ref):
    a_ps = pstate_ref[0, 0]
    t_ps = pstate_ref[0, 1]
    hwrite = t_ps + jax.nn.gelu(jnp.dot(a_ps, wt_ref[...]) + bt_ref[...])
    ard = ae_ref[0] + (rw_ref[0] + noise_ref[0]) + de_ref[0]
    v = jnp.dot(hwrite, wwrh_ref[...]) + jnp.dot(ard, wwre_ref[...])
    keys = keys_ref[0]
    wide = old_ref[0]
    sub = jnp.bitwise_and(xwc_ref[0], G - 1)
    old = jnp.zeros((L, MN), jnp.float32)
    for g in range(G):
        old = old + jnp.where(sub == g, wide[:, g * MN:(g + 1) * MN], 0.0)
    lo = lax.dot_general(old, keys, (((1,), (1,)), ((), ()))) * INV_SQRT_MN
    ln = lax.dot_general(v, keys, (((1,), (1,)), ((), ()))) * INV_SQRT_MN
    eq = xwc_ref[0] == xwr_ref[0]
    later = (lax.broadcasted_iota(jnp.int32, (L, L), 1)
             > lax.broadcasted_iota(jnp.int32, (L, L), 0))
    dup = jnp.any(eq & later, axis=1, keepdims=True)
    valid = jnp.where(dup, 0.0, 1.0)
    elo = jnp.exp(lo) * valid
    eln = jnp.exp(ln) * valid
    seg = seg_ref[0]
    accg = accg_ref[0]
    se16 = jnp.zeros((1, NQ), jnp.float32)
    acc16 = jnp.zeros((NQ, MN), jnp.float32)
    for g in range(G):
        se16 = se16 + seg[:, g * NQ:(g + 1) * NQ]
        acc16 = acc16 + accg[g * NQ:(g + 1) * NQ, g * MN:(g + 1) * MN]
    se = se16 + jnp.sum(eln - elo, axis=0, keepdims=True)
    acc = (acc16
           + lax.dot_general(eln, v, (((0,), (0,)), ((), ())))
           - lax.dot_general(elo, old, (((0,), (0,)), ((), ()))))
    eye = (lax.broadcasted_iota(jnp.int32, (NQ, NQ), 0)
           == lax.broadcasted_iota(jnp.int32, (NQ, NQ), 1))
    se_col = jnp.sum(jnp.where(eye, se, 0.0), axis=1, keepdims=True)
    reads = acc / se_col
    y = (jnp.dot(hread_ref[0], wo1_ref[...])
         + jnp.dot(reads[0:L], wo2a_ref[...])
         + jnp.dot(reads[L:NQ], wo2b_ref[...]))
    y_ref[0] = y


def _epi_call(accg, seg, hread, keys, pstate, ae3, de3, old3, rw_col, noise3,
              xw_row, xw_col, Wt, bt2, Wwrh, Wwre, Wo1, Wo2a, Wo2b,
              interpret=False):
    const = lambda *blk: pl.BlockSpec(blk, lambda b: (0,) * len(blk))
    perb = lambda *blk: pl.BlockSpec(blk, lambda b: (b,) + (0,) * (len(blk) - 1))
    return pl.pallas_call(
        _epi_body,
        grid=(B,),
        in_specs=[
            perb(1, GQ, GM),
            perb(1, 1, GQ),
            perb(1, L, H),
            perb(1, NQ, MN),
            perb(1, 2, L, H),
            perb(1, L, E),
            perb(1, L, E),
            perb(1, L, 128),
            perb(1, L, 1),
            perb(1, L, E),
            perb(1, 1, L),
            perb(1, L, 1),
            const(H, H),
            const(1, H),
            const(H, MN),
            const(E, MN),
            const(H, H),
            const(MN, H),
            const(MN, H),
        ],
        out_specs=perb(1, L, H),
        out_shape=jax.ShapeDtypeStruct((B, L, H), jnp.float32),
        compiler_params=pltpu.CompilerParams(
            dimension_semantics=("arbitrary",)),
        interpret=interpret,
    )(accg, seg, hread, keys, pstate, ae3, de3, old3, rw_col, noise3,
      xw_row, xw_col, Wt, bt2, Wwrh, Wwre, Wo1, Wo2a, Wo2b)


def kernel(state, pstate, ac, rw, dn, x_w, step, params, past,
           Wt, bt, aw, dw, W_wr, W_rk, W_o):
    noise = 0.001 * jax.random.normal(
        jax.random.key(1), (B, L, E), dtype=jnp.float32)
    past2d = past.reshape(B * MS // G, GM)
    bt2 = bt.reshape(1, H)
    ae_f, de_f, old_f = _sc_gather(
        aw, dw, past2d,
        ac.reshape(B * L), dn.reshape(B * L), x_w.reshape(B * L))
    accg, seg, hread, keys = _stream_call(
        past2d, state, Wt, bt2, W_rk[:H, 0:MN], W_rk[:H, MN:2 * MN])
    y = _epi_call(
        accg, seg, hread, keys, pstate,
        ae_f.reshape(B, L, E), de_f.reshape(B, L, E),
        old_f.reshape(B, L, 128),
        rw.reshape(B, L, 1), noise,
        x_w.reshape(B, 1, L), x_w.reshape(B, L, 1),
        Wt, bt2, W_wr[:H], W_wr[H:],
        W_o[:H], W_o[H + E:H + E + MN], W_o[H + E + MN:])
    return y


# speedup vs baseline: 1.0563x; 1.0176x over previous
import functools

import jax
import jax.numpy as jnp
import numpy as np
from jax import lax
from jax.experimental import pallas as pl
from jax.experimental.pallas import tpu as pltpu
from jax.experimental.pallas import tpu_sc as plsc

B, L = 16, 8
H = 256
E = 128
MN = 32
MS = 65536
NR = 2
NQ = L * NR
G = 4
CHUNK = 32768
CS = CHUNK // G
HCS = CS // 2
NCHUNK = MS // CHUNK
INV_SQRT_MN = 1.0 / float(np.sqrt(32.0))
GQ = G * NQ
GM = G * MN


def _sc_gather(aw, dw, past2d, ac_flat, dn_flat, xw_flat):
    mesh = plsc.VectorSubcoreMesh(core_axis_name="c", subcore_axis_name="s")

    @functools.partial(
        pl.kernel,
        mesh=mesh,
        out_type=[
            jax.ShapeDtypeStruct((B * L, E), jnp.float32),
            jax.ShapeDtypeStruct((B * L, E), jnp.float32),
            jax.ShapeDtypeStruct((B * L, 128), jnp.float32),
        ],
        scratch_types=[
            pltpu.VMEM((16,), jnp.int32),
            pltpu.VMEM((16,), jnp.int32),
            pltpu.VMEM((16, E), jnp.float32),
            pltpu.SemaphoreType.DMA,
        ],
    )
    def k(aw_h, dw_h, past_h, ac_h, dn_h, xw_h, ae_o, de_o, old_o,
          idx_v, idx2_v, rows_v, sem):
        c = lax.axis_index("c")
        s = lax.axis_index("s")
        wid = s * 2 + c
        grp = wid // 8
        base = pl.multiple_of((wid % 8) * 16, 16)

        @pl.when(grp == 0)
        def _():
            pltpu.sync_copy(ac_h.at[pl.ds(base, 16)], idx_v)
            pltpu.async_copy(aw_h.at[idx_v], rows_v, sem).wait()
            pltpu.sync_copy(rows_v, ae_o.at[pl.ds(base, 16)])

        @pl.when(grp == 1)
        def _():
            pltpu.sync_copy(dn_h.at[pl.ds(base, 16)], idx_v)
            pltpu.async_copy(dw_h.at[idx_v], rows_v, sem).wait()
            pltpu.sync_copy(rows_v, de_o.at[pl.ds(base, 16)])

        @pl.when(grp == 2)
        def _():
            pltpu.sync_copy(xw_h.at[pl.ds(base, 16)], idx_v)
            xv = idx_v[...]
            half = lax.shift_right_logical(
                lax.broadcasted_iota(jnp.int32, (16,), 0), 3)
            brow = (base // 8) + half
            idx2_v[...] = lax.shift_right_logical(xv + brow * MS, 2)
            pltpu.async_copy(past_h.at[idx2_v], rows_v, sem).wait()
            pltpu.sync_copy(rows_v, old_o.at[pl.ds(base, 16)])

    return k(aw, dw, past2d, ac_flat, dn_flat, xw_flat)


def _stream_body(past_ref, state_ref, wt_ref, bt_ref, wrk0_ref, wrk1_ref,
                 pstate_ref, ae_ref, de_ref, old_ref, rw_ref, noise_ref,
                 xwr_ref, xwc_ref, wwrh_ref, wwre_ref,
                 wo1_ref, wo2a_ref, wo2b_ref,
                 y_ref, bd_s, acc_a, acc_b, se_s, hread_s, keys_s):
    i = pl.program_id(1)

    @pl.when(i == 0)
    def _():
        a_state = state_ref[0, 0]
        t_state = state_ref[0, 1]
        hread = t_state + jax.nn.gelu(
            jnp.dot(a_state, wt_ref[...]) + bt_ref[...])
        hread_s[...] = hread
        keys = jnp.concatenate(
            [jnp.dot(hread, wrk0_ref[...]),
             jnp.dot(hread, wrk1_ref[...])], axis=0)
        keys_s[...] = keys
        kt = jnp.transpose(keys)
        t1 = jnp.concatenate([kt] * G, axis=0)
        t2 = jnp.concatenate([t1] * G, axis=1)
        rowg = lax.broadcasted_iota(jnp.int32, (GM, GQ), 0) // MN
        colg = lax.broadcasted_iota(jnp.int32, (GM, GQ), 1) // NQ
        bd_s[...] = jnp.where(rowg == colg, t2, 0.0)
        acc_a[...] = jnp.zeros_like(acc_a)
        acc_b[...] = jnp.zeros_like(acc_b)
        se_s[...] = jnp.zeros_like(se_s)

    bd = bd_s[...]
    ca = past_ref[0:HCS]
    cb = past_ref[HCS:CS]
    la = jnp.dot(ca, bd)
    lb = jnp.dot(cb, bd)
    pa = jnp.exp(la * INV_SQRT_MN)
    pb = jnp.exp(lb * INV_SQRT_MN)
    se_s[...] += (jnp.sum(pa, axis=0, keepdims=True)
                  + jnp.sum(pb, axis=0, keepdims=True))
    acc_a[...] += lax.dot_general(pa, ca, (((0,), (0,)), ((), ())))
    acc_b[...] += lax.dot_general(pb, cb, (((0,), (0,)), ((), ())))

    @pl.when(i == NCHUNK - 1)
    def _():
        a_ps = pstate_ref[0, 0]
        t_ps = pstate_ref[0, 1]
        hwrite = t_ps + jax.nn.gelu(jnp.dot(a_ps, wt_ref[...]) + bt_ref[...])
        ard = ae_ref[0] + (rw_ref[0] + noise_ref[0]) + de_ref[0]
        v = jnp.dot(hwrite, wwrh_ref[...]) + jnp.dot(ard, wwre_ref[...])
        keys = keys_s[...]
        wide = old_ref[0]
        sub = jnp.bitwise_and(xwc_ref[0], G - 1)
        old = jnp.zeros((L, MN), jnp.float32)
        for g in range(G):
            old = old + jnp.where(sub == g, wide[:, g * MN:(g + 1) * MN], 0.0)
        lo = lax.dot_general(old, keys, (((1,), (1,)), ((), ()))) * INV_SQRT_MN
        ln = lax.dot_general(v, keys, (((1,), (1,)), ((), ()))) * INV_SQRT_MN
        eq = xwc_ref[0] == xwr_ref[0]
        later = (lax.broadcasted_iota(jnp.int32, (L, L), 1)
                 > lax.broadcasted_iota(jnp.int32, (L, L), 0))
        dup = jnp.any(eq & later, axis=1, keepdims=True)
        valid = jnp.where(dup, 0.0, 1.0)
        elo = jnp.exp(lo) * valid
        eln = jnp.exp(ln) * valid
        seg = se_s[...]
        accg = acc_a[...] + acc_b[...]
        se16 = jnp.zeros((1, NQ), jnp.float32)
        acc16 = jnp.zeros((NQ, MN), jnp.float32)
        for g in range(G):
            se16 = se16 + seg[:, g * NQ:(g + 1) * NQ]
            acc16 = acc16 + accg[g * NQ:(g + 1) * NQ, g * MN:(g + 1) * MN]
        se = se16 + jnp.sum(eln - elo, axis=0, keepdims=True)
        acc = (acc16
               + lax.dot_general(eln, v, (((0,), (0,)), ((), ())))
               - lax.dot_general(elo, old, (((0,), (0,)), ((), ()))))
        eye = (lax.broadcasted_iota(jnp.int32, (NQ, NQ), 0)
               == lax.broadcasted_iota(jnp.int32, (NQ, NQ), 1))
        se_col = jnp.sum(jnp.where(eye, se, 0.0), axis=1, keepdims=True)
        reads = acc / se_col
        y = (jnp.dot(hread_s[...], wo1_ref[...])
             + jnp.dot(reads[0:L], wo2a_ref[...])
             + jnp.dot(reads[L:NQ], wo2b_ref[...]))
        y_ref[0] = y


def _stream_call(past2d, state, Wt, bt2, Wrk0, Wrk1, pstate, ae3, de3,
                 old3, rw_col, noise3, xw_row, xw_col, Wwrh, Wwre,
                 Wo1, Wo2a, Wo2b, interpret=False):
    const = lambda *blk: pl.BlockSpec(blk, lambda b, i: (0,) * len(blk))
    perb = lambda *blk: pl.BlockSpec(blk, lambda b, i: (b,) + (0,) * (len(blk) - 1))
    return pl.pallas_call(
        _stream_body,
        grid=(B, NCHUNK),
        in_specs=[
            pl.BlockSpec((CS, GM), lambda b, i: (b * NCHUNK + i, 0)),
            perb(1, 2, L, H),
            const(H, H),
            const(1, H),
            const(H, MN),
            const(H, MN),
            perb(1, 2, L, H),
            perb(1, L, E),
            perb(1, L, E),
            perb(1, L, 128),
            perb(1, L, 1),
            perb(1, L, E),
            perb(1, 1, L),
            perb(1, L, 1),
            const(H, MN),
            const(E, MN),
            const(H, H),
            const(MN, H),
            const(MN, H),
        ],
        out_specs=perb(1, L, H),
        out_shape=jax.ShapeDtypeStruct((B, L, H), jnp.float32),
        scratch_shapes=[
            pltpu.VMEM((GM, GQ), jnp.float32),
            pltpu.VMEM((GQ, GM), jnp.float32),
            pltpu.VMEM((GQ, GM), jnp.float32),
            pltpu.VMEM((1, GQ), jnp.float32),
            pltpu.VMEM((L, H), jnp.float32),
            pltpu.VMEM((NQ, MN), jnp.float32),
        ],
        compiler_params=pltpu.CompilerParams(
            dimension_semantics=("parallel", "arbitrary")),
        interpret=interpret,
    )(past2d, state, Wt, bt2, Wrk0, Wrk1, pstate, ae3, de3, old3,
      rw_col, noise3, xw_row, xw_col, Wwrh, Wwre, Wo1, Wo2a, Wo2b)


def kernel(state, pstate, ac, rw, dn, x_w, step, params, past,
           Wt, bt, aw, dw, W_wr, W_rk, W_o):
    noise = 0.001 * jax.random.normal(
        jax.random.key(1), (B, L, E), dtype=jnp.float32)
    past2d = past.reshape(B * MS // G, GM)
    bt2 = bt.reshape(1, H)
    ae_f, de_f, old_f = _sc_gather(
        aw, dw, past2d,
        ac.reshape(B * L), dn.reshape(B * L), x_w.reshape(B * L))
    y = _stream_call(
        past2d, state, Wt, bt2, W_rk[:H, 0:MN], W_rk[:H, MN:2 * MN],
        pstate, ae_f.reshape(B, L, E), de_f.reshape(B, L, E),
        old_f.reshape(B, L, 128),
        rw.reshape(B, L, 1), noise,
        x_w.reshape(B, 1, L), x_w.reshape(B, L, 1),
        W_wr[:H], W_wr[H:],
        W_o[:H], W_o[H + E:H + E + MN], W_o[H + E + MN:])
    return y
